# baseline (speedup 1.0000x reference)
import jax
import jax.numpy as jnp
from jax import lax
from jax.experimental import pallas as pl
from jax.experimental.pallas import tpu as pltpu
from jax.experimental.pallas import tpu_sc as plsc

B, T, D = 16, 2048, 1024


def _laststep_body(payload_hbm, lens_hbm, out_hbm, lens_s, sems):
    cid = lax.axis_index("c")

    @pl.when(cid == 0)
    def _():
        pltpu.sync_copy(lens_hbm, lens_s)
        copies = []
        for b in range(B):
            row = (lens_s[b] - 1) & (T - 1)
            copies.append(
                pltpu.async_copy(
                    payload_hbm.at[b * T + row], out_hbm.at[b], sems.at[b]
                )
            )
        for c in copies:
            c.wait()


def kernel(payload, seq_lens):
    flat = payload.reshape(B * T, D)
    mesh = plsc.ScalarSubcoreMesh(axis_name="c", num_cores=1)
    f = pl.kernel(
        _laststep_body,
        mesh=mesh,
        out_type=jax.ShapeDtypeStruct((B, D), jnp.float32),
        scratch_types=[
            pltpu.SMEM((B,), jnp.int32),
            pltpu.SemaphoreType.DMA((B,)),
        ],
    )
    return f(flat, seq_lens.astype(jnp.int32))
